# Initial kernel scaffold; baseline (speedup 1.0000x reference)
import functools
import math

import jax
import jax.numpy as jnp
from jax import lax
from jax.experimental import pallas as pl
from jax.experimental.pallas import tpu as pltpu
from jax.experimental.pallas import tpu_sc as plsc

VOCAB_N = 1000000
DIM = 32
ROWS = 4096
COLS = 200
SCALE = math.sqrt(float(DIM))

NW = 32
B_TOTAL = ROWS * COLS
B_PER_W = B_TOTAL // NW
CHUNK = 128
NCHUNK = B_PER_W // CHUNK
NBUF = 4
NGRP = NCHUNK // NBUF

_mesh = plsc.VectorSubcoreMesh(core_axis_name="c", subcore_axis_name="s")


def _scale_chunk(src, dst):
    def body(i, carry):
        for u in range(8):
            r = i * 8 + u
            for h in (0, 16):
                dst[r, pl.ds(h, 16)] = src[r, pl.ds(h, 16)] * SCALE
        return carry

    lax.fori_loop(0, CHUNK // 8, body, 0)


@functools.partial(
    pl.kernel,
    out_type=jax.ShapeDtypeStruct((B_TOTAL, DIM), jnp.float32),
    mesh=_mesh,
    scratch_types=[
        pltpu.VMEM((NCHUNK, CHUNK), jnp.int32),
        *[pltpu.VMEM((CHUNK, DIM), jnp.float32) for _ in range(NBUF)],
        *[pltpu.VMEM((CHUNK, DIM), jnp.float32) for _ in range(NBUF)],
        *[pltpu.SemaphoreType.DMA for _ in range(NBUF)],
        *[pltpu.SemaphoreType.DMA for _ in range(NBUF)],
    ],
)
def _emb_lookup(x_hbm, table_hbm, out_hbm, idx_v, *bufs_and_sems):
    in_bufs = bufs_and_sems[:NBUF]
    out_bufs = bufs_and_sems[NBUF:2 * NBUF]
    gsems = bufs_and_sems[2 * NBUF:3 * NBUF]
    ssems = bufs_and_sems[3 * NBUF:4 * NBUF]

    wid = lax.axis_index("s") * 2 + lax.axis_index("c")
    row_base = wid * B_PER_W

    pltpu.sync_copy(x_hbm.at[wid], idx_v)

    for b in range(NBUF):
        pltpu.async_copy(table_hbm.at[idx_v.at[b]], in_bufs[b], gsems[b])

    def group(g, carry):
        for b in range(NBUF):
            j = g * NBUF + b
            pltpu.make_async_copy(table_hbm.at[idx_v.at[j]], in_bufs[b],
                                  gsems[b]).wait()
            @pl.when(g > 0)
            def _():
                pltpu.make_async_copy(
                    out_bufs[b],
                    out_hbm.at[pl.ds(row_base + (j - NBUF) * CHUNK, CHUNK)],
                    ssems[b]).wait()

            _scale_chunk(in_bufs[b], out_bufs[b])

            @pl.when(g < NGRP - 1)
            def _():
                pltpu.async_copy(table_hbm.at[idx_v.at[j + NBUF]], in_bufs[b],
                                 gsems[b])

            pltpu.async_copy(
                out_bufs[b],
                out_hbm.at[pl.ds(row_base + j * CHUNK, CHUNK)],
                ssems[b])
        return carry

    lax.fori_loop(0, NGRP, group, 0)

    for b in range(NBUF):
        j = (NGRP - 1) * NBUF + b
        pltpu.make_async_copy(
            out_bufs[b],
            out_hbm.at[pl.ds(row_base + j * CHUNK, CHUNK)],
            ssems[b]).wait()


def kernel(x, table):
    x3 = x.astype(jnp.int32).reshape(NW, NCHUNK, CHUNK)
    out = _emb_lookup(x3, table)
    return out.reshape(ROWS, COLS, DIM)


# speedup vs baseline: 1.4756x; 1.4756x over previous
import functools
import math

import jax
import jax.numpy as jnp
from jax import lax
from jax.experimental import pallas as pl
from jax.experimental.pallas import tpu as pltpu
from jax.experimental.pallas import tpu_sc as plsc

VOCAB_N = 1000000
DIM = 32
ROWS = 4096
COLS = 200
SCALE = math.sqrt(float(DIM))

NW = 32
B_TOTAL = ROWS * COLS
B_PER_W = B_TOTAL // NW
CHUNK = 128
NCHUNK = B_PER_W // CHUNK
NBUF = 4
NGRP = NCHUNK // NBUF

_mesh = plsc.VectorSubcoreMesh(core_axis_name="c", subcore_axis_name="s")


def _scale_chunk(src, dst):
    def body(i, carry):
        for u in range(8):
            r = i * 8 + u
            for h in (0, 16):
                dst[r, pl.ds(h, 16)] = src[r, pl.ds(h, 16)] * SCALE
        return carry

    lax.fori_loop(0, CHUNK // 8, body, 0)


@functools.partial(
    pl.kernel,
    out_type=jax.ShapeDtypeStruct((B_TOTAL, DIM), jnp.float32),
    mesh=_mesh,
    compiler_params=pltpu.CompilerParams(use_tc_tiling_on_sc=False),
    scratch_types=[
        pltpu.VMEM((NCHUNK, CHUNK), jnp.int32),
        *[pltpu.VMEM((CHUNK, DIM), jnp.float32) for _ in range(NBUF)],
        *[pltpu.VMEM((CHUNK, DIM), jnp.float32) for _ in range(NBUF)],
        *[pltpu.SemaphoreType.DMA for _ in range(NBUF)],
        *[pltpu.SemaphoreType.DMA for _ in range(NBUF)],
    ],
)
def _emb_lookup(x_hbm, table_hbm, out_hbm, idx_v, *bufs_and_sems):
    in_bufs = bufs_and_sems[:NBUF]
    out_bufs = bufs_and_sems[NBUF:2 * NBUF]
    gsems = bufs_and_sems[2 * NBUF:3 * NBUF]
    ssems = bufs_and_sems[3 * NBUF:4 * NBUF]

    wid = lax.axis_index("s") * 2 + lax.axis_index("c")
    row_base = wid * B_PER_W

    pltpu.sync_copy(x_hbm.at[wid], idx_v)

    for b in range(NBUF):
        pltpu.async_copy(table_hbm.at[idx_v.at[b]], in_bufs[b], gsems[b])

    def group(g, carry):
        for b in range(NBUF):
            j = g * NBUF + b
            pltpu.make_async_copy(table_hbm.at[idx_v.at[j]], in_bufs[b],
                                  gsems[b]).wait()
            @pl.when(g > 0)
            def _():
                pltpu.make_async_copy(
                    out_bufs[b],
                    out_hbm.at[pl.ds(row_base + (j - NBUF) * CHUNK, CHUNK)],
                    ssems[b]).wait()

            _scale_chunk(in_bufs[b], out_bufs[b])

            @pl.when(g < NGRP - 1)
            def _():
                pltpu.async_copy(table_hbm.at[idx_v.at[j + NBUF]], in_bufs[b],
                                 gsems[b])

            pltpu.async_copy(
                out_bufs[b],
                out_hbm.at[pl.ds(row_base + j * CHUNK, CHUNK)],
                ssems[b])
        return carry

    lax.fori_loop(0, NGRP, group, 0)

    for b in range(NBUF):
        j = (NGRP - 1) * NBUF + b
        pltpu.make_async_copy(
            out_bufs[b],
            out_hbm.at[pl.ds(row_base + j * CHUNK, CHUNK)],
            ssems[b]).wait()


def kernel(x, table):
    x3 = x.astype(jnp.int32).reshape(NW, NCHUNK, CHUNK)
    out = _emb_lookup(x3, table)
    return out.reshape(ROWS, COLS, DIM)


# speedup vs baseline: 1.4833x; 1.0052x over previous
import functools
import math

import jax
import jax.numpy as jnp
from jax import lax
from jax.experimental import pallas as pl
from jax.experimental.pallas import tpu as pltpu
from jax.experimental.pallas import tpu_sc as plsc

VOCAB_N = 1000000
DIM = 32
ROWS = 4096
COLS = 200
SCALE = math.sqrt(float(DIM))

NW = 32
ROWS_PER_W = ROWS // NW
CHUNK = COLS
NCHUNK = ROWS_PER_W
NBUF = 4
NGRP = NCHUNK // NBUF

_mesh = plsc.VectorSubcoreMesh(core_axis_name="c", subcore_axis_name="s")


def _scale_chunk(src, dst):
    def body(i, carry):
        for u in range(4):
            r = i * 4 + u
            for h in (0, 16):
                dst[r, pl.ds(h, 16)] = src[r, pl.ds(h, 16)] * SCALE
        return carry

    lax.fori_loop(0, CHUNK // 4, body, 0)


@functools.partial(
    pl.kernel,
    out_type=jax.ShapeDtypeStruct((ROWS, COLS, DIM), jnp.float32),
    mesh=_mesh,
    compiler_params=pltpu.CompilerParams(use_tc_tiling_on_sc=False),
    scratch_types=[
        pltpu.VMEM((ROWS_PER_W, COLS), jnp.int32),
        *[pltpu.VMEM((CHUNK, DIM), jnp.float32) for _ in range(NBUF)],
        *[pltpu.VMEM((CHUNK, DIM), jnp.float32) for _ in range(NBUF)],
        *[pltpu.SemaphoreType.DMA for _ in range(NBUF)],
        *[pltpu.SemaphoreType.DMA for _ in range(NBUF)],
    ],
)
def _emb_lookup(x_hbm, table_hbm, out_hbm, idx_v, *bufs_and_sems):
    in_bufs = bufs_and_sems[:NBUF]
    out_bufs = bufs_and_sems[NBUF:2 * NBUF]
    gsems = bufs_and_sems[2 * NBUF:3 * NBUF]
    ssems = bufs_and_sems[3 * NBUF:4 * NBUF]

    wid = lax.axis_index("s") * 2 + lax.axis_index("c")
    row_base = wid * ROWS_PER_W

    pltpu.sync_copy(x_hbm.at[pl.ds(row_base, ROWS_PER_W)], idx_v)

    def _idx_slice(j):
        return idx_v.at[j]

    def _out_slice(j):
        return out_hbm.at[row_base + j]

    for b in range(NBUF):
        pltpu.async_copy(table_hbm.at[_idx_slice(b)], in_bufs[b], gsems[b])

    def group(g, carry):
        for b in range(NBUF):
            j = g * NBUF + b
            pltpu.make_async_copy(table_hbm.at[_idx_slice(j)], in_bufs[b],
                                  gsems[b]).wait()
            @pl.when(g > 0)
            def _():
                pltpu.make_async_copy(out_bufs[b], _out_slice(j - NBUF),
                                      ssems[b]).wait()

            _scale_chunk(in_bufs[b], out_bufs[b])

            @pl.when(g < NGRP - 1)
            def _():
                pltpu.async_copy(table_hbm.at[_idx_slice(j + NBUF)],
                                 in_bufs[b], gsems[b])

            pltpu.async_copy(out_bufs[b], _out_slice(j), ssems[b])
        return carry

    lax.fori_loop(0, NGRP, group, 0)

    for b in range(NBUF):
        j = (NGRP - 1) * NBUF + b
        pltpu.make_async_copy(out_bufs[b], _out_slice(j), ssems[b]).wait()


def kernel(x, table):
    return _emb_lookup(x.astype(jnp.int32), table)
